# baseline (speedup 1.0000x reference)
import functools

import jax
import jax.numpy as jnp
from jax import lax
from jax.experimental import pallas as pl
from jax.experimental.pallas import tpu as pltpu
from jax.experimental.pallas import tpu_sc as plsc

N = 10000
E = 160000
D = 256
H = D // 2
NC = 2
NS = 16
LANES = 16
NPAD = 10112
EPAD = 163840
EPT = EPAD // NS
CHUNK = 32
NCHUNKS = EPT // CHUNK
MACRO = 8
NM = NCHUNKS // MACRO
BODIES = NCHUNKS // 2
ROWS_PER_SUB = NPAD // NS
GBYTES = 4 * CHUNK * H * 4
DUMP = NPAD - 1

_f32 = jnp.float32


def _proj_body(e_ref, ws_ref, wm_ref, wd_ref, bm_ref, bsum_ref,
               cb_ref, sd_ref):
    e = e_ref[...]
    ps = jnp.dot(e, ws_ref[...], preferred_element_type=_f32)
    pm = jnp.dot(e, wm_ref[...], preferred_element_type=_f32) + bm_ref[...]
    pd = jnp.dot(e, wd_ref[...], preferred_element_type=_f32) + bsum_ref[...]
    cb_ref[0] = jnp.concatenate([ps[:, :H], pm[:, :H]], axis=-1)
    cb_ref[1] = jnp.concatenate([ps[:, H:], pm[:, H:]], axis=-1)
    sd_ref[0] = pd[:, :H]
    sd_ref[1] = pd[:, H:]


def _node_proj(e_pad, w_src, w_msg, w_dst, b_msg, b_sum):
    rb = NPAD // 16
    grid = (NPAD // rb,)
    full = pl.BlockSpec((D, D), lambda i: (0, 0))
    vec = pl.BlockSpec((1, D), lambda i: (0, 0))
    return pl.pallas_call(
        _proj_body,
        grid=grid,
        in_specs=[pl.BlockSpec((rb, D), lambda i: (i, 0)), full, full, full,
                  vec, vec],
        out_specs=[pl.BlockSpec((2, rb, D), lambda i: (0, i, 0)),
                   pl.BlockSpec((2, rb, H), lambda i: (0, i, 0))],
        out_shape=[jax.ShapeDtypeStruct((2, NPAD, D), _f32),
                   jax.ShapeDtypeStruct((2, NPAD, H), _f32)],
    )(e_pad, w_src, w_msg, w_dst, b_msg, b_sum)


def _ang_body(a_ref, w_ref, g_ref):
    g = jnp.dot(a_ref[...], w_ref[...], preferred_element_type=_f32)
    g_ref[0] = g[:, :H]
    g_ref[1] = g[:, H:]


def _ang_proj(a, w_ang):
    rb = 2000
    grid = (E // rb,)
    return pl.pallas_call(
        _ang_body,
        grid=grid,
        in_specs=[pl.BlockSpec((rb, D), lambda i: (i, 0)),
                  pl.BlockSpec((D, D), lambda i: (0, 0))],
        out_specs=pl.BlockSpec((2, rb, H), lambda i: (0, i, 0)),
        out_shape=jax.ShapeDtypeStruct((2, E, H), _f32),
    )(a, w_ang)


def _edge_body(cb_t, sd_t, g_t, srco, dsto, dstp, zeros_hbm, agg_out,
               so0, do0, dp0, so1, do1, dp1,
               cbA, sdA, gA, cbB, sdB, gB, m_v,
               agg_sh, semA, semB):
    cid = lax.axis_index("c")
    sid = lax.axis_index("s")

    my_rows = pl.ds(sid * ROWS_PER_SUB, ROWS_PER_SUB)
    pltpu.sync_copy(zeros_hbm.at[my_rows], agg_sh.at[my_rows])

    idx_row0 = sid * (EPT // CHUNK)

    def load_macro(m, so, do, dp):
        rb = pl.multiple_of(idx_row0 + m * MACRO, 8)
        pltpu.sync_copy(srco.at[cid, pl.ds(rb, MACRO)], so)
        pltpu.sync_copy(dsto.at[cid, pl.ds(rb, MACRO)], do)
        pltpu.sync_copy(dstp.at[pl.ds(rb, MACRO)], dp)

    def issue(c, cb_b, sd_b, g_b, sem, so, do):
        r = lax.rem(c, MACRO)
        pltpu.async_copy(cb_t.at[so.at[r]], cb_b, sem)
        pltpu.async_copy(sd_t.at[do.at[r]], sd_b, sem)
        gbase = pl.multiple_of(
            cid * E + jnp.minimum(sid * EPT + c * CHUNK, E - CHUNK), 8)
        pltpu.async_copy(g_t.at[pl.ds(gbase, CHUNK)], g_b, sem)

    def issue_p(c, cb_b, sd_b, g_b, sem):
        par = lax.rem(lax.div(c, MACRO), 2)

        @pl.when(par == 0)
        def _():
            issue(c, cb_b, sd_b, g_b, sem, so0, do0)

        @pl.when(par == 1)
        def _():
            issue(c, cb_b, sd_b, g_b, sem, so1, do1)

    def drain(cb_b, sd_b, g_b, sem):
        pltpu.make_async_copy(cb_t.at[pl.ds(0, CHUNK)], cb_b, sem).wait()
        dummy = sd_t.at[pl.ds(0, CHUNK)]
        pltpu.make_async_copy(dummy, sd_b, sem).wait()
        pltpu.make_async_copy(dummy, g_b, sem).wait()

    def compute(cb_b, sd_b, g_b):
        @plsc.parallel_loop(0, CHUNK, unroll=2)
        def _(i):
            for j in range(H // LANES):
                fs = pl.ds(j * LANES, LANES)
                x = cb_b[i, fs] + sd_b[i, fs] + g_b[i, fs]
                m_v[i, fs] = cb_b[i, pl.ds(H + j * LANES, LANES)] / (
                    1.0 + jnp.exp(x))

    def scatter(c):
        r = lax.rem(c, MACRO)
        par = lax.rem(lax.div(c, MACRO), 2)

        @pl.when(par == 0)
        def _():
            pltpu.sync_copy(m_v, agg_sh.at[dp0.at[r]], add=True)

        @pl.when(par == 1)
        def _():
            pltpu.sync_copy(m_v, agg_sh.at[dp1.at[r]], add=True)

    load_macro(0, so0, do0, dp0)
    issue(0, cbA, sdA, gA, semA, so0, do0)

    def body(k, carry):
        c0 = 2 * k
        c1 = c0 + 1
        cn = c0 + 2

        issue_p(c1, cbB, sdB, gB, semB)

        @pl.when(lax.rem(k, MACRO // 2) == 0)
        def _():
            mn = jnp.minimum(lax.div(k, MACRO // 2) + 1, NM - 1)

            @pl.when(lax.rem(mn, 2) == 0)
            def _():
                load_macro(mn, so0, do0, dp0)

            @pl.when(lax.rem(mn, 2) == 1)
            def _():
                load_macro(mn, so1, do1, dp1)

        drain(cbA, sdA, gA, semA)
        compute(cbA, sdA, gA)
        scatter(c0)

        @pl.when(cn < NCHUNKS)
        def _():
            issue_p(cn, cbA, sdA, gA, semA)

        drain(cbB, sdB, gB, semB)
        compute(cbB, sdB, gB)
        scatter(c1)
        return carry

    lax.fori_loop(0, BODIES, body, 0)
    plsc.subcore_barrier()

    pltpu.sync_copy(agg_sh.at[my_rows], agg_out.at[cid, my_rows])


_edge_phase = functools.partial(
    pl.kernel,
    _edge_body,
    out_type=jax.ShapeDtypeStruct((2, NPAD, H), _f32),
    mesh=plsc.VectorSubcoreMesh(core_axis_name="c", subcore_axis_name="s"),
    scratch_types=[
        pltpu.VMEM((MACRO, CHUNK), jnp.int32),
        pltpu.VMEM((MACRO, CHUNK), jnp.int32),
        pltpu.VMEM((MACRO, CHUNK), jnp.int32),
        pltpu.VMEM((MACRO, CHUNK), jnp.int32),
        pltpu.VMEM((MACRO, CHUNK), jnp.int32),
        pltpu.VMEM((MACRO, CHUNK), jnp.int32),
        pltpu.VMEM((CHUNK, D), _f32),
        pltpu.VMEM((CHUNK, H), _f32),
        pltpu.VMEM((CHUNK, H), _f32),
        pltpu.VMEM((CHUNK, D), _f32),
        pltpu.VMEM((CHUNK, H), _f32),
        pltpu.VMEM((CHUNK, H), _f32),
        pltpu.VMEM((CHUNK, H), _f32),
        pltpu.VMEM_SHARED((NPAD, H), _f32),
        pltpu.SemaphoreType.DMA,
        pltpu.SemaphoreType.DMA,
    ],
)()


def _mlp_body(e_ref, a0_ref, a1_ref, w1e_ref, w1a0_ref, w1a1_ref, b1_ref,
              w2_ref, b2_ref, gam_ref, bet_ref, out_ref):
    e = e_ref[...]
    h = (jnp.dot(e, w1e_ref[...], preferred_element_type=_f32)
         + jnp.dot(a0_ref[...], w1a0_ref[...], preferred_element_type=_f32)
         + jnp.dot(a1_ref[...], w1a1_ref[...], preferred_element_type=_f32)
         + b1_ref[...])
    h = h * (1.0 / (1.0 + jnp.exp(-h)))
    h = jnp.dot(h, w2_ref[...], preferred_element_type=_f32) + b2_ref[...]
    x = e + h
    mean = jnp.mean(x, axis=-1, keepdims=True)
    cen = x - mean
    var = jnp.mean(cen * cen, axis=-1, keepdims=True)
    out_ref[...] = cen * lax.rsqrt(var + 1e-5) * gam_ref[...] + bet_ref[...]


def _node_mlp(e, agg0, agg1, w1e, w1a0, w1a1, b1, w2, b2, gamma, beta):
    rb = 1000
    grid = (N // rb,)
    vec = pl.BlockSpec((1, D), lambda i: (0, 0))
    return pl.pallas_call(
        _mlp_body,
        grid=grid,
        in_specs=[pl.BlockSpec((rb, D), lambda i: (i, 0)),
                  pl.BlockSpec((rb, H), lambda i: (i, 0)),
                  pl.BlockSpec((rb, H), lambda i: (i, 0)),
                  pl.BlockSpec((D, D), lambda i: (0, 0)),
                  pl.BlockSpec((H, D), lambda i: (0, 0)),
                  pl.BlockSpec((H, D), lambda i: (0, 0)),
                  vec,
                  pl.BlockSpec((D, D), lambda i: (0, 0)),
                  vec, vec, vec],
        out_specs=pl.BlockSpec((rb, D), lambda i: (i, 0)),
        out_shape=jax.ShapeDtypeStruct((N, D), _f32),
    )(e, agg0, agg1, w1e, w1a0, w1a1, b1, w2, b2, gamma, beta)


def kernel(e, a, edge_index, W_src, b_src, W_dst, b_dst, W_ang, b_ang,
           W_msg, b_msg, W1, b1, W2, b2, gamma, beta):
    ei = edge_index.astype(jnp.int32)
    src, dst = ei[0], ei[1]
    nl2e = jnp.float32(-1.0)
    b_sum = ((b_src + b_dst + b_ang) * nl2e).reshape(1, D)

    e_pad = jnp.concatenate([e, jnp.zeros((NPAD - N, D), _f32)])
    cb_t, sd_t = _node_proj(
        e_pad, W_src * nl2e, W_msg, W_dst * nl2e, b_msg.reshape(1, D), b_sum)
    g_t = _ang_proj(a, W_ang * nl2e).reshape(2 * E, H)

    src_p = jnp.concatenate([src, jnp.zeros((EPAD - E,), jnp.int32)])
    dst_p = jnp.concatenate([dst, jnp.full((EPAD - E,), DUMP, jnp.int32)])
    srco = jnp.stack([src_p, src_p + NPAD]).reshape(2, EPAD // CHUNK, CHUNK)
    dsto = jnp.stack([dst_p, dst_p + NPAD]).reshape(2, EPAD // CHUNK, CHUNK)
    dstp = dst_p.reshape(EPAD // CHUNK, CHUNK)

    zeros = jnp.zeros((NPAD, H), _f32)
    agg = _edge_phase(cb_t.reshape(2 * NPAD, D), sd_t.reshape(2 * NPAD, H),
                      g_t, srco, dsto, dstp, zeros)

    return _node_mlp(e, agg[0, :N], agg[1, :N], W1[:D], W1[D:D + H],
                     W1[D + H:], b1.reshape(1, D), W2, b2.reshape(1, D),
                     gamma.reshape(1, D), beta.reshape(1, D))


# speedup vs baseline: 1.0669x; 1.0669x over previous
import functools

import jax
import jax.numpy as jnp
from jax import lax
from jax.experimental import pallas as pl
from jax.experimental.pallas import tpu as pltpu
from jax.experimental.pallas import tpu_sc as plsc

N = 10000
E = 160000
D = 256
H = D // 2
NC = 2
NS = 16
LANES = 16
NPAD = 10112
EPAD = 163840
EPT = EPAD // NS
CHUNK = 32
NCHUNKS = EPT // CHUNK
MACRO = 8
NM = NCHUNKS // MACRO
BODIES = NCHUNKS // 2
ROWS_PER_SUB = NPAD // NS
GBYTES = 4 * CHUNK * H * 4
DUMP = NPAD - 1

_f32 = jnp.float32


def _pack_half(x):
    u = lax.bitcast_convert_type(x.astype(jnp.bfloat16), jnp.uint16)
    u = u.astype(jnp.uint32)
    return (jnp.left_shift(u[:, H // 2:], jnp.uint32(16))
            | u[:, :H // 2]).astype(jnp.int32)


def _proj_body(e_ref, ws_ref, wm_ref, wd_ref, bm_ref, bsum_ref,
               sg_ref, sm_ref, sd_ref):
    e = e_ref[...]
    ps = jnp.dot(e, ws_ref[...], preferred_element_type=_f32)
    pm = jnp.dot(e, wm_ref[...], preferred_element_type=_f32) + bm_ref[...]
    pd = jnp.dot(e, wd_ref[...], preferred_element_type=_f32) + bsum_ref[...]
    sg_ref[0] = ps[:, :H]
    sg_ref[1] = ps[:, H:]
    sm_ref[0] = pm[:, :H]
    sm_ref[1] = pm[:, H:]
    sd_ref[0] = pd[:, :H]
    sd_ref[1] = pd[:, H:]


def _node_proj(e_pad, w_src, w_msg, w_dst, b_msg, b_sum):
    rb = NPAD // 16
    grid = (NPAD // rb,)
    full = pl.BlockSpec((D, D), lambda i: (0, 0))
    vec = pl.BlockSpec((1, D), lambda i: (0, 0))
    return pl.pallas_call(
        _proj_body,
        grid=grid,
        in_specs=[pl.BlockSpec((rb, D), lambda i: (i, 0)), full, full, full,
                  vec, vec],
        out_specs=[pl.BlockSpec((2, rb, H), lambda i: (0, i, 0))] * 3,
        out_shape=[jax.ShapeDtypeStruct((2, NPAD, H), _f32)] * 3,
    )(e_pad, w_src, w_msg, w_dst, b_msg, b_sum)


def _ang_body(a_ref, w_ref, g_ref):
    g = jnp.dot(a_ref[...], w_ref[...], preferred_element_type=_f32)
    g_ref[0] = _pack_half(g[:, :H])
    g_ref[1] = _pack_half(g[:, H:])


def _ang_proj(a, w_ang):
    rb = 2000
    grid = (E // rb,)
    return pl.pallas_call(
        _ang_body,
        grid=grid,
        in_specs=[pl.BlockSpec((rb, D), lambda i: (i, 0)),
                  pl.BlockSpec((D, D), lambda i: (0, 0))],
        out_specs=pl.BlockSpec((2, rb, H // 2), lambda i: (0, i, 0)),
        out_shape=jax.ShapeDtypeStruct((2, E, H // 2), jnp.int32),
    )(a, w_ang)


def _edge_body(sg_t, sm_t, sd_t, g_t, srco, dsto, dstp, zeros_hbm, agg_out,
               so0, do0, dp0, so1, do1, dp1,
               sgA, smA, sdA, gA, sgB, smB, sdB, gB, m_v,
               agg_sh, semA, semB):
    cid = lax.axis_index("c")
    sid = lax.axis_index("s")

    my_rows = pl.ds(sid * ROWS_PER_SUB, ROWS_PER_SUB)
    pltpu.sync_copy(zeros_hbm.at[my_rows], agg_sh.at[my_rows])

    idx_row0 = sid * (EPT // CHUNK)

    def load_macro(m, so, do, dp):
        rb = pl.multiple_of(idx_row0 + m * MACRO, 8)
        pltpu.sync_copy(srco.at[cid, pl.ds(rb, MACRO)], so)
        pltpu.sync_copy(dsto.at[cid, pl.ds(rb, MACRO)], do)
        pltpu.sync_copy(dstp.at[pl.ds(rb, MACRO)], dp)

    def issue(c, sg_b, sm_b, sd_b, g_b, sem, so, do):
        r = lax.rem(c, MACRO)
        pltpu.async_copy(sg_t.at[so.at[r]], sg_b, sem)
        pltpu.async_copy(sm_t.at[so.at[r]], sm_b, sem)
        pltpu.async_copy(sd_t.at[do.at[r]], sd_b, sem)
        gbase = pl.multiple_of(
            cid * E + jnp.minimum(sid * EPT + c * CHUNK, E - CHUNK), 8)
        pltpu.async_copy(g_t.at[pl.ds(gbase, CHUNK)], g_b, sem)

    def issue_p(c, sg_b, sm_b, sd_b, g_b, sem):
        par = lax.rem(lax.div(c, MACRO), 2)

        @pl.when(par == 0)
        def _():
            issue(c, sg_b, sm_b, sd_b, g_b, sem, so0, do0)

        @pl.when(par == 1)
        def _():
            issue(c, sg_b, sm_b, sd_b, g_b, sem, so1, do1)

    def drain(sg_b, sm_b, sd_b, g_b, sem):
        pltpu.make_async_copy(sg_t.at[pl.ds(0, CHUNK)], sg_b, sem).wait()
        pltpu.make_async_copy(sm_t.at[pl.ds(0, CHUNK)], sm_b, sem).wait()
        pltpu.make_async_copy(sd_t.at[pl.ds(0, CHUNK)], sd_b, sem).wait()
        pltpu.make_async_copy(g_t.at[pl.ds(0, CHUNK)], g_b, sem).wait()

    def compute(sg_b, sm_b, sd_b, g_b):
        m_hi = jnp.int32(-65536)
        sh = jnp.int32(16)
        bc = lambda v: lax.bitcast_convert_type(v, _f32)

        @plsc.parallel_loop(0, CHUNK, unroll=2)
        def _(i):
            for k in range(H // 2 // LANES):
                ks = pl.ds(k * LANES, LANES)
                fhi = pl.ds(H // 2 + k * LANES, LANES)
                wg = g_b[i, ks]
                lo_x = sg_b[i, ks] + sd_b[i, ks] + bc(lax.shift_left(wg, sh))
                hi_x = sg_b[i, fhi] + sd_b[i, fhi] + bc(wg & m_hi)
                m_v[i, ks] = sm_b[i, ks] / (1.0 + jnp.exp(lo_x))
                m_v[i, fhi] = sm_b[i, fhi] / (1.0 + jnp.exp(hi_x))

    def scatter(c):
        r = lax.rem(c, MACRO)
        par = lax.rem(lax.div(c, MACRO), 2)

        @pl.when(par == 0)
        def _():
            pltpu.sync_copy(m_v, agg_sh.at[dp0.at[r]], add=True)

        @pl.when(par == 1)
        def _():
            pltpu.sync_copy(m_v, agg_sh.at[dp1.at[r]], add=True)

    load_macro(0, so0, do0, dp0)
    issue(0, sgA, smA, sdA, gA, semA, so0, do0)

    def body(k, carry):
        c0 = 2 * k
        c1 = c0 + 1
        cn = c0 + 2

        issue_p(c1, sgB, smB, sdB, gB, semB)

        @pl.when(lax.rem(k, MACRO // 2) == 0)
        def _():
            mn = jnp.minimum(lax.div(k, MACRO // 2) + 1, NM - 1)

            @pl.when(lax.rem(mn, 2) == 0)
            def _():
                load_macro(mn, so0, do0, dp0)

            @pl.when(lax.rem(mn, 2) == 1)
            def _():
                load_macro(mn, so1, do1, dp1)

        drain(sgA, smA, sdA, gA, semA)
        compute(sgA, smA, sdA, gA)
        scatter(c0)

        @pl.when(cn < NCHUNKS)
        def _():
            issue_p(cn, sgA, smA, sdA, gA, semA)

        drain(sgB, smB, sdB, gB, semB)
        compute(sgB, smB, sdB, gB)
        scatter(c1)
        return carry

    lax.fori_loop(0, BODIES, body, 0)
    plsc.subcore_barrier()

    pltpu.sync_copy(agg_sh.at[my_rows], agg_out.at[cid, my_rows])


_edge_phase = functools.partial(
    pl.kernel,
    _edge_body,
    out_type=jax.ShapeDtypeStruct((2, NPAD, H), _f32),
    mesh=plsc.VectorSubcoreMesh(core_axis_name="c", subcore_axis_name="s"),
    scratch_types=[
        pltpu.VMEM((MACRO, CHUNK), jnp.int32),
        pltpu.VMEM((MACRO, CHUNK), jnp.int32),
        pltpu.VMEM((MACRO, CHUNK), jnp.int32),
        pltpu.VMEM((MACRO, CHUNK), jnp.int32),
        pltpu.VMEM((MACRO, CHUNK), jnp.int32),
        pltpu.VMEM((MACRO, CHUNK), jnp.int32),
        pltpu.VMEM((CHUNK, H), _f32),
        pltpu.VMEM((CHUNK, H), _f32),
        pltpu.VMEM((CHUNK, H), _f32),
        pltpu.VMEM((CHUNK, H // 2), jnp.int32),
        pltpu.VMEM((CHUNK, H), _f32),
        pltpu.VMEM((CHUNK, H), _f32),
        pltpu.VMEM((CHUNK, H), _f32),
        pltpu.VMEM((CHUNK, H // 2), jnp.int32),
        pltpu.VMEM((CHUNK, H), _f32),
        pltpu.VMEM_SHARED((NPAD, H), _f32),
        pltpu.SemaphoreType.DMA,
        pltpu.SemaphoreType.DMA,
    ],
)()


def _mlp_body(e_ref, a0_ref, a1_ref, w1e_ref, w1a0_ref, w1a1_ref, b1_ref,
              w2_ref, b2_ref, gam_ref, bet_ref, out_ref):
    e = e_ref[...]
    h = (jnp.dot(e, w1e_ref[...], preferred_element_type=_f32)
         + jnp.dot(a0_ref[...], w1a0_ref[...], preferred_element_type=_f32)
         + jnp.dot(a1_ref[...], w1a1_ref[...], preferred_element_type=_f32)
         + b1_ref[...])
    h = h * (1.0 / (1.0 + jnp.exp(-h)))
    h = jnp.dot(h, w2_ref[...], preferred_element_type=_f32) + b2_ref[...]
    x = e + h
    mean = jnp.mean(x, axis=-1, keepdims=True)
    cen = x - mean
    var = jnp.mean(cen * cen, axis=-1, keepdims=True)
    out_ref[...] = cen * lax.rsqrt(var + 1e-5) * gam_ref[...] + bet_ref[...]


def _node_mlp(e, agg0, agg1, w1e, w1a0, w1a1, b1, w2, b2, gamma, beta):
    rb = 1000
    grid = (N // rb,)
    vec = pl.BlockSpec((1, D), lambda i: (0, 0))
    return pl.pallas_call(
        _mlp_body,
        grid=grid,
        in_specs=[pl.BlockSpec((rb, D), lambda i: (i, 0)),
                  pl.BlockSpec((rb, H), lambda i: (i, 0)),
                  pl.BlockSpec((rb, H), lambda i: (i, 0)),
                  pl.BlockSpec((D, D), lambda i: (0, 0)),
                  pl.BlockSpec((H, D), lambda i: (0, 0)),
                  pl.BlockSpec((H, D), lambda i: (0, 0)),
                  vec,
                  pl.BlockSpec((D, D), lambda i: (0, 0)),
                  vec, vec, vec],
        out_specs=pl.BlockSpec((rb, D), lambda i: (i, 0)),
        out_shape=jax.ShapeDtypeStruct((N, D), _f32),
    )(e, agg0, agg1, w1e, w1a0, w1a1, b1, w2, b2, gamma, beta)


def kernel(e, a, edge_index, W_src, b_src, W_dst, b_dst, W_ang, b_ang,
           W_msg, b_msg, W1, b1, W2, b2, gamma, beta):
    ei = edge_index.astype(jnp.int32)
    src, dst = ei[0], ei[1]
    nl2e = jnp.float32(-1.0)
    b_sum = ((b_src + b_dst + b_ang) * nl2e).reshape(1, D)

    e_pad = jnp.concatenate([e, jnp.zeros((NPAD - N, D), _f32)])
    sg_t, sm_t, sd_t = _node_proj(
        e_pad, W_src * nl2e, W_msg, W_dst * nl2e, b_msg.reshape(1, D), b_sum)
    g_t = _ang_proj(a, W_ang * nl2e).reshape(2 * E, H // 2)

    src_p = jnp.concatenate([src, jnp.zeros((EPAD - E,), jnp.int32)])
    dst_p = jnp.concatenate([dst, jnp.full((EPAD - E,), DUMP, jnp.int32)])
    srco = jnp.stack([src_p, src_p + NPAD]).reshape(2, EPAD // CHUNK, CHUNK)
    dsto = jnp.stack([dst_p, dst_p + NPAD]).reshape(2, EPAD // CHUNK, CHUNK)
    dstp = dst_p.reshape(EPAD // CHUNK, CHUNK)

    zeros = jnp.zeros((NPAD, H), _f32)
    agg = _edge_phase(sg_t.reshape(2 * NPAD, H), sm_t.reshape(2 * NPAD, H),
                      sd_t.reshape(2 * NPAD, H), g_t,
                      srco, dsto, dstp, zeros)

    return _node_mlp(e, agg[0, :N], agg[1, :N], W1[:D], W1[D:D + H],
                     W1[D + H:], b1.reshape(1, D), W2, b2.reshape(1, D),
                     gamma.reshape(1, D), beta.reshape(1, D))


# speedup vs baseline: 1.0785x; 1.0109x over previous
import functools

import jax
import jax.numpy as jnp
from jax import lax
from jax.experimental import pallas as pl
from jax.experimental.pallas import tpu as pltpu
from jax.experimental.pallas import tpu_sc as plsc

N = 10000
E = 160000
D = 256
H = D // 2
NC = 2
NS = 16
LANES = 16
NPAD = 10112
EPAD = 163840
EPT = EPAD // NS
CHUNK = 32
NCHUNKS = EPT // CHUNK
MACRO = 8
NM = NCHUNKS // MACRO
BODIES = NCHUNKS // 2
ROWS_PER_SUB = NPAD // NS
GBYTES = 4 * CHUNK * H * 4
DUMP = NPAD - 1

_f32 = jnp.float32


def _proj_body(e_ref, ws_ref, wm_ref, wd_ref, bm_ref, bsum_ref,
               sg_ref, sm_ref, sd_ref):
    e = e_ref[...]
    ps = jnp.dot(e, ws_ref[...], preferred_element_type=_f32)
    pm = jnp.dot(e, wm_ref[...], preferred_element_type=_f32) + bm_ref[...]
    pd = jnp.dot(e, wd_ref[...], preferred_element_type=_f32) + bsum_ref[...]
    sg_ref[0] = ps[:, :H]
    sg_ref[1] = ps[:, H:]
    sm_ref[0] = pm[:, :H]
    sm_ref[1] = pm[:, H:]
    sd_ref[0] = pd[:, :H]
    sd_ref[1] = pd[:, H:]


def _node_proj(e_pad, w_src, w_msg, w_dst, b_msg, b_sum):
    rb = NPAD // 16
    grid = (NPAD // rb,)
    full = pl.BlockSpec((D, D), lambda i: (0, 0))
    vec = pl.BlockSpec((1, D), lambda i: (0, 0))
    return pl.pallas_call(
        _proj_body,
        grid=grid,
        in_specs=[pl.BlockSpec((rb, D), lambda i: (i, 0)), full, full, full,
                  vec, vec],
        out_specs=[pl.BlockSpec((2, rb, H), lambda i: (0, i, 0))] * 3,
        out_shape=[jax.ShapeDtypeStruct((2, NPAD, H), _f32)] * 3,
    )(e_pad, w_src, w_msg, w_dst, b_msg, b_sum)


def _ang_body(a_ref, w_ref, g_ref):
    g = jnp.dot(a_ref[...].astype(jnp.bfloat16),
                w_ref[...].astype(jnp.bfloat16),
                preferred_element_type=_f32)
    g_ref[0] = g[:, :H]
    g_ref[1] = g[:, H:]


def _ang_proj(a, w_ang):
    rb = 2000
    grid = (E // rb,)
    return pl.pallas_call(
        _ang_body,
        grid=grid,
        in_specs=[pl.BlockSpec((rb, D), lambda i: (i, 0)),
                  pl.BlockSpec((D, D), lambda i: (0, 0))],
        out_specs=pl.BlockSpec((2, rb, H), lambda i: (0, i, 0)),
        out_shape=jax.ShapeDtypeStruct((2, E, H), _f32),
    )(a, w_ang)


def _edge_body(sg_t, sm_t, sd_t, g_t, srco, dsto, dstp, zeros_hbm, agg_out,
               so0, do0, dp0, so1, do1, dp1,
               sgA, smA, sdA, gA, sgB, smB, sdB, gB, m_v,
               agg_sh, semA, semB):
    cid = lax.axis_index("c")
    sid = lax.axis_index("s")

    my_rows = pl.ds(sid * ROWS_PER_SUB, ROWS_PER_SUB)
    pltpu.sync_copy(zeros_hbm.at[my_rows], agg_sh.at[my_rows])

    idx_row0 = sid * (EPT // CHUNK)

    def load_macro(m, so, do, dp):
        rb = pl.multiple_of(idx_row0 + m * MACRO, 8)
        pltpu.sync_copy(srco.at[cid, pl.ds(rb, MACRO)], so)
        pltpu.sync_copy(dsto.at[cid, pl.ds(rb, MACRO)], do)
        pltpu.sync_copy(dstp.at[pl.ds(rb, MACRO)], dp)

    def issue(c, sg_b, sm_b, sd_b, g_b, sem, so, do):
        r = lax.rem(c, MACRO)
        pltpu.async_copy(sg_t.at[so.at[r]], sg_b, sem)
        pltpu.async_copy(sm_t.at[so.at[r]], sm_b, sem)
        pltpu.async_copy(sd_t.at[do.at[r]], sd_b, sem)
        gbase = pl.multiple_of(
            cid * E + jnp.minimum(sid * EPT + c * CHUNK, E - CHUNK), 8)
        pltpu.async_copy(g_t.at[pl.ds(gbase, CHUNK)], g_b, sem)

    def issue_p(c, sg_b, sm_b, sd_b, g_b, sem):
        par = lax.rem(lax.div(c, MACRO), 2)

        @pl.when(par == 0)
        def _():
            issue(c, sg_b, sm_b, sd_b, g_b, sem, so0, do0)

        @pl.when(par == 1)
        def _():
            issue(c, sg_b, sm_b, sd_b, g_b, sem, so1, do1)

    def drain(sg_b, sm_b, sd_b, g_b, sem):
        dummy = sg_t.at[pl.ds(0, CHUNK)]
        pltpu.make_async_copy(dummy, sg_b, sem).wait()
        pltpu.make_async_copy(dummy, sm_b, sem).wait()
        pltpu.make_async_copy(dummy, sd_b, sem).wait()
        pltpu.make_async_copy(dummy, g_b, sem).wait()

    def compute(sg_b, sm_b, sd_b, g_b):
        @plsc.parallel_loop(0, CHUNK, unroll=2)
        def _(i):
            for j in range(H // LANES):
                fs = pl.ds(j * LANES, LANES)
                x = sg_b[i, fs] + sd_b[i, fs] + g_b[i, fs]
                m_v[i, fs] = sm_b[i, fs] / (1.0 + jnp.exp(x))

    def scatter(c):
        r = lax.rem(c, MACRO)
        par = lax.rem(lax.div(c, MACRO), 2)

        @pl.when(par == 0)
        def _():
            pltpu.sync_copy(m_v, agg_sh.at[dp0.at[r]], add=True)

        @pl.when(par == 1)
        def _():
            pltpu.sync_copy(m_v, agg_sh.at[dp1.at[r]], add=True)

    load_macro(0, so0, do0, dp0)
    issue(0, sgA, smA, sdA, gA, semA, so0, do0)

    def body(k, carry):
        c0 = 2 * k
        c1 = c0 + 1
        cn = c0 + 2

        issue_p(c1, sgB, smB, sdB, gB, semB)

        @pl.when(lax.rem(k, MACRO // 2) == 0)
        def _():
            mn = jnp.minimum(lax.div(k, MACRO // 2) + 1, NM - 1)

            @pl.when(lax.rem(mn, 2) == 0)
            def _():
                load_macro(mn, so0, do0, dp0)

            @pl.when(lax.rem(mn, 2) == 1)
            def _():
                load_macro(mn, so1, do1, dp1)

        drain(sgA, smA, sdA, gA, semA)
        compute(sgA, smA, sdA, gA)
        scatter(c0)

        @pl.when(cn < NCHUNKS)
        def _():
            issue_p(cn, sgA, smA, sdA, gA, semA)

        drain(sgB, smB, sdB, gB, semB)
        compute(sgB, smB, sdB, gB)
        scatter(c1)
        return carry

    lax.fori_loop(0, BODIES, body, 0)
    plsc.subcore_barrier()

    pltpu.sync_copy(agg_sh.at[my_rows], agg_out.at[cid, my_rows])


_edge_phase = functools.partial(
    pl.kernel,
    _edge_body,
    out_type=jax.ShapeDtypeStruct((2, NPAD, H), _f32),
    mesh=plsc.VectorSubcoreMesh(core_axis_name="c", subcore_axis_name="s"),
    scratch_types=[
        pltpu.VMEM((MACRO, CHUNK), jnp.int32),
        pltpu.VMEM((MACRO, CHUNK), jnp.int32),
        pltpu.VMEM((MACRO, CHUNK), jnp.int32),
        pltpu.VMEM((MACRO, CHUNK), jnp.int32),
        pltpu.VMEM((MACRO, CHUNK), jnp.int32),
        pltpu.VMEM((MACRO, CHUNK), jnp.int32),
        pltpu.VMEM((CHUNK, H), _f32),
        pltpu.VMEM((CHUNK, H), _f32),
        pltpu.VMEM((CHUNK, H), _f32),
        pltpu.VMEM((CHUNK, H), _f32),
        pltpu.VMEM((CHUNK, H), _f32),
        pltpu.VMEM((CHUNK, H), _f32),
        pltpu.VMEM((CHUNK, H), _f32),
        pltpu.VMEM((CHUNK, H), _f32),
        pltpu.VMEM((CHUNK, H), _f32),
        pltpu.VMEM_SHARED((NPAD, H), _f32),
        pltpu.SemaphoreType.DMA,
        pltpu.SemaphoreType.DMA,
    ],
)()


def _mlp_body(e_ref, a0_ref, a1_ref, w1e_ref, w1a0_ref, w1a1_ref, b1_ref,
              w2_ref, b2_ref, gam_ref, bet_ref, out_ref):
    e = e_ref[...]
    h = (jnp.dot(e, w1e_ref[...], preferred_element_type=_f32)
         + jnp.dot(a0_ref[...], w1a0_ref[...], preferred_element_type=_f32)
         + jnp.dot(a1_ref[...], w1a1_ref[...], preferred_element_type=_f32)
         + b1_ref[...])
    h = h * (1.0 / (1.0 + jnp.exp(-h)))
    h = jnp.dot(h, w2_ref[...], preferred_element_type=_f32) + b2_ref[...]
    x = e + h
    mean = jnp.mean(x, axis=-1, keepdims=True)
    cen = x - mean
    var = jnp.mean(cen * cen, axis=-1, keepdims=True)
    out_ref[...] = cen * lax.rsqrt(var + 1e-5) * gam_ref[...] + bet_ref[...]


def _node_mlp(e, agg0, agg1, w1e, w1a0, w1a1, b1, w2, b2, gamma, beta):
    rb = 1000
    grid = (N // rb,)
    vec = pl.BlockSpec((1, D), lambda i: (0, 0))
    return pl.pallas_call(
        _mlp_body,
        grid=grid,
        in_specs=[pl.BlockSpec((rb, D), lambda i: (i, 0)),
                  pl.BlockSpec((rb, H), lambda i: (i, 0)),
                  pl.BlockSpec((rb, H), lambda i: (i, 0)),
                  pl.BlockSpec((D, D), lambda i: (0, 0)),
                  pl.BlockSpec((H, D), lambda i: (0, 0)),
                  pl.BlockSpec((H, D), lambda i: (0, 0)),
                  vec,
                  pl.BlockSpec((D, D), lambda i: (0, 0)),
                  vec, vec, vec],
        out_specs=pl.BlockSpec((rb, D), lambda i: (i, 0)),
        out_shape=jax.ShapeDtypeStruct((N, D), _f32),
    )(e, agg0, agg1, w1e, w1a0, w1a1, b1, w2, b2, gamma, beta)


def kernel(e, a, edge_index, W_src, b_src, W_dst, b_dst, W_ang, b_ang,
           W_msg, b_msg, W1, b1, W2, b2, gamma, beta):
    ei = edge_index.astype(jnp.int32)
    src, dst = ei[0], ei[1]
    nl2e = jnp.float32(-1.0)
    b_sum = ((b_src + b_dst + b_ang) * nl2e).reshape(1, D)

    sg_t, sm_t, sd_t = _node_proj(
        e, W_src * nl2e, W_msg, W_dst * nl2e, b_msg.reshape(1, D), b_sum)
    g_t = _ang_proj(a, W_ang * nl2e).reshape(2 * E, H)

    src_p = jnp.concatenate([src, jnp.zeros((EPAD - E,), jnp.int32)])
    dst_p = jnp.concatenate([dst, jnp.full((EPAD - E,), DUMP, jnp.int32)])
    srco = jnp.stack([src_p, src_p + NPAD]).reshape(2, EPAD // CHUNK, CHUNK)
    dsto = jnp.stack([dst_p, dst_p + NPAD]).reshape(2, EPAD // CHUNK, CHUNK)
    dstp = dst_p.reshape(EPAD // CHUNK, CHUNK)

    zeros = jnp.zeros((NPAD, H), _f32)
    agg = _edge_phase(sg_t.reshape(2 * NPAD, H), sm_t.reshape(2 * NPAD, H),
                      sd_t.reshape(2 * NPAD, H), g_t,
                      srco, dsto, dstp, zeros)

    return _node_mlp(e, agg[0, :N], agg[1, :N], W1[:D], W1[D:D + H],
                     W1[D + H:], b1.reshape(1, D), W2, b2.reshape(1, D),
                     gamma.reshape(1, D), beta.reshape(1, D))


# speedup vs baseline: 1.0939x; 1.0143x over previous
import functools

import jax
import jax.numpy as jnp
from jax import lax
from jax.experimental import pallas as pl
from jax.experimental.pallas import tpu as pltpu
from jax.experimental.pallas import tpu_sc as plsc

N = 10000
E = 160000
D = 256
H = D // 2
NC = 2
NS = 16
LANES = 16
NPAD = 10112
EPAD = 163840
EPT = EPAD // NS
CHUNK = 32
NCHUNKS = EPT // CHUNK
MACRO = 8
NM = NCHUNKS // MACRO
BODIES = NCHUNKS // 2
ROWS_PER_SUB = NPAD // NS
GBYTES = 4 * CHUNK * H * 4
DUMP = NPAD - 1

_f32 = jnp.float32


def _proj_body(e_ref, ws_ref, wm_ref, wd_ref, bm_ref, bsum_ref,
               sg_ref, sm_ref, sd_ref):
    e = e_ref[...]
    ps = jnp.dot(e, ws_ref[...], preferred_element_type=_f32)
    pm = jnp.dot(e, wm_ref[...], preferred_element_type=_f32) + bm_ref[...]
    pd = jnp.dot(e, wd_ref[...], preferred_element_type=_f32) + bsum_ref[...]
    sg_ref[0] = ps[:, :H]
    sg_ref[1] = ps[:, H:]
    sm_ref[0] = pm[:, :H]
    sm_ref[1] = pm[:, H:]
    sd_ref[0] = pd[:, :H]
    sd_ref[1] = pd[:, H:]


def _node_proj(e_pad, w_src, w_msg, w_dst, b_msg, b_sum):
    rb = NPAD // 16
    grid = (NPAD // rb,)
    full = pl.BlockSpec((D, D), lambda i: (0, 0))
    vec = pl.BlockSpec((1, D), lambda i: (0, 0))
    return pl.pallas_call(
        _proj_body,
        grid=grid,
        in_specs=[pl.BlockSpec((rb, D), lambda i: (i, 0)), full, full, full,
                  vec, vec],
        out_specs=[pl.BlockSpec((2, rb, H), lambda i: (0, i, 0))] * 3,
        out_shape=[jax.ShapeDtypeStruct((2, NPAD, H), _f32)] * 3,
    )(e_pad, w_src, w_msg, w_dst, b_msg, b_sum)


def _ang_body(a_ref, w_ref, g_ref):
    g = jnp.dot(a_ref[...], w_ref[...], preferred_element_type=_f32)
    g_ref[0] = g[:, :H]
    g_ref[1] = g[:, H:]


def _ang_proj(a, w_ang):
    rb = 2000
    grid = (E // rb,)
    return pl.pallas_call(
        _ang_body,
        grid=grid,
        in_specs=[pl.BlockSpec((rb, D), lambda i: (i, 0)),
                  pl.BlockSpec((D, D), lambda i: (0, 0))],
        out_specs=pl.BlockSpec((2, rb, H), lambda i: (0, i, 0)),
        out_shape=jax.ShapeDtypeStruct((2, E, H), _f32),
    )(a, w_ang)


def _edge_body(sg_t, sm_t, sd_t, g_t, srco, dsto, dstp, zeros_hbm, agg_out,
               so0, do0, dp0, so1, do1, dp1,
               sgA, smA, sdA, gA, sgB, smB, sdB, gB, m_v,
               agg_sh, semA, semB):
    cid = lax.axis_index("c")
    sid = lax.axis_index("s")

    my_rows = pl.ds(sid * ROWS_PER_SUB, ROWS_PER_SUB)
    pltpu.sync_copy(zeros_hbm.at[my_rows], agg_sh.at[my_rows])

    idx_row0 = sid * (EPT // CHUNK)

    def load_macro(m, so, do, dp):
        rb = pl.multiple_of(idx_row0 + m * MACRO, 8)
        pltpu.sync_copy(srco.at[cid, pl.ds(rb, MACRO)], so)
        pltpu.sync_copy(dsto.at[cid, pl.ds(rb, MACRO)], do)
        pltpu.sync_copy(dstp.at[pl.ds(rb, MACRO)], dp)

    def issue(c, sg_b, sm_b, sd_b, g_b, sem, so, do):
        r = lax.rem(c, MACRO)
        pltpu.async_copy(sg_t.at[so.at[r]], sg_b, sem)
        pltpu.async_copy(sm_t.at[so.at[r]], sm_b, sem)
        pltpu.async_copy(sd_t.at[do.at[r]], sd_b, sem)
        gbase = pl.multiple_of(
            cid * E + jnp.minimum(sid * EPT + c * CHUNK, E - CHUNK), 8)
        pltpu.async_copy(g_t.at[pl.ds(gbase, CHUNK)], g_b, sem)

    def issue_p(c, sg_b, sm_b, sd_b, g_b, sem):
        par = lax.rem(lax.div(c, MACRO), 2)

        @pl.when(par == 0)
        def _():
            issue(c, sg_b, sm_b, sd_b, g_b, sem, so0, do0)

        @pl.when(par == 1)
        def _():
            issue(c, sg_b, sm_b, sd_b, g_b, sem, so1, do1)

    def drain(sg_b, sm_b, sd_b, g_b, sem):
        dummy = sg_t.at[pl.ds(0, CHUNK)]
        pltpu.make_async_copy(dummy, sg_b, sem).wait()
        pltpu.make_async_copy(dummy, sm_b, sem).wait()
        pltpu.make_async_copy(dummy, sd_b, sem).wait()
        pltpu.make_async_copy(dummy, g_b, sem).wait()

    def compute(sg_b, sm_b, sd_b, g_b):
        @plsc.parallel_loop(0, CHUNK, unroll=2)
        def _(i):
            for j in range(H // LANES):
                fs = pl.ds(j * LANES, LANES)
                x = sg_b[i, fs] + sd_b[i, fs] + g_b[i, fs]
                m_v[i, fs] = sm_b[i, fs] / (1.0 + jnp.exp(x))

    def scatter(c):
        r = lax.rem(c, MACRO)
        par = lax.rem(lax.div(c, MACRO), 2)

        @pl.when(par == 0)
        def _():
            pltpu.sync_copy(m_v, agg_sh.at[dp0.at[r]], add=True)

        @pl.when(par == 1)
        def _():
            pltpu.sync_copy(m_v, agg_sh.at[dp1.at[r]], add=True)

    load_macro(0, so0, do0, dp0)
    issue(0, sgA, smA, sdA, gA, semA, so0, do0)

    def body(k, carry):
        c0 = 2 * k
        c1 = c0 + 1
        cn = c0 + 2

        issue_p(c1, sgB, smB, sdB, gB, semB)

        @pl.when(lax.rem(k, MACRO // 2) == 0)
        def _():
            mn = jnp.minimum(lax.div(k, MACRO // 2) + 1, NM - 1)

            @pl.when(lax.rem(mn, 2) == 0)
            def _():
                load_macro(mn, so0, do0, dp0)

            @pl.when(lax.rem(mn, 2) == 1)
            def _():
                load_macro(mn, so1, do1, dp1)

        drain(sgA, smA, sdA, gA, semA)
        compute(sgA, smA, sdA, gA)
        scatter(c0)

        @pl.when(cn < NCHUNKS)
        def _():
            issue_p(cn, sgA, smA, sdA, gA, semA)

        drain(sgB, smB, sdB, gB, semB)
        compute(sgB, smB, sdB, gB)
        scatter(c1)
        return carry

    lax.fori_loop(0, BODIES, body, 0)
    plsc.subcore_barrier()

    pltpu.sync_copy(agg_sh.at[my_rows], agg_out.at[cid, my_rows])


_edge_phase = functools.partial(
    pl.kernel,
    _edge_body,
    out_type=jax.ShapeDtypeStruct((2, NPAD, H), _f32),
    mesh=plsc.VectorSubcoreMesh(core_axis_name="c", subcore_axis_name="s"),
    scratch_types=[
        pltpu.VMEM((MACRO, CHUNK), jnp.int32),
        pltpu.VMEM((MACRO, CHUNK), jnp.int32),
        pltpu.VMEM((MACRO, CHUNK), jnp.int32),
        pltpu.VMEM((MACRO, CHUNK), jnp.int32),
        pltpu.VMEM((MACRO, CHUNK), jnp.int32),
        pltpu.VMEM((MACRO, CHUNK), jnp.int32),
        pltpu.VMEM((CHUNK, H), _f32),
        pltpu.VMEM((CHUNK, H), _f32),
        pltpu.VMEM((CHUNK, H), _f32),
        pltpu.VMEM((CHUNK, H), _f32),
        pltpu.VMEM((CHUNK, H), _f32),
        pltpu.VMEM((CHUNK, H), _f32),
        pltpu.VMEM((CHUNK, H), _f32),
        pltpu.VMEM((CHUNK, H), _f32),
        pltpu.VMEM((CHUNK, H), _f32),
        pltpu.VMEM_SHARED((NPAD, H), _f32),
        pltpu.SemaphoreType.DMA,
        pltpu.SemaphoreType.DMA,
    ],
)()


def _mlp_body(e_ref, a0_ref, a1_ref, w1e_ref, w1a0_ref, w1a1_ref, b1_ref,
              w2_ref, b2_ref, gam_ref, bet_ref, out_ref):
    e = e_ref[...]
    h = (jnp.dot(e, w1e_ref[...], preferred_element_type=_f32)
         + jnp.dot(a0_ref[...], w1a0_ref[...], preferred_element_type=_f32)
         + jnp.dot(a1_ref[...], w1a1_ref[...], preferred_element_type=_f32)
         + b1_ref[...])
    h = h * (1.0 / (1.0 + jnp.exp(-h)))
    h = jnp.dot(h, w2_ref[...], preferred_element_type=_f32) + b2_ref[...]
    x = e + h
    mean = jnp.mean(x, axis=-1, keepdims=True)
    cen = x - mean
    var = jnp.mean(cen * cen, axis=-1, keepdims=True)
    out_ref[...] = cen * lax.rsqrt(var + 1e-5) * gam_ref[...] + bet_ref[...]


def _node_mlp(e, agg0, agg1, w1e, w1a0, w1a1, b1, w2, b2, gamma, beta):
    rb = 1000
    grid = (N // rb,)
    vec = pl.BlockSpec((1, D), lambda i: (0, 0))
    return pl.pallas_call(
        _mlp_body,
        grid=grid,
        in_specs=[pl.BlockSpec((rb, D), lambda i: (i, 0)),
                  pl.BlockSpec((rb, H), lambda i: (i, 0)),
                  pl.BlockSpec((rb, H), lambda i: (i, 0)),
                  pl.BlockSpec((D, D), lambda i: (0, 0)),
                  pl.BlockSpec((H, D), lambda i: (0, 0)),
                  pl.BlockSpec((H, D), lambda i: (0, 0)),
                  vec,
                  pl.BlockSpec((D, D), lambda i: (0, 0)),
                  vec, vec, vec],
        out_specs=pl.BlockSpec((rb, D), lambda i: (i, 0)),
        out_shape=jax.ShapeDtypeStruct((N, D), _f32),
    )(e, agg0, agg1, w1e, w1a0, w1a1, b1, w2, b2, gamma, beta)


def kernel(e, a, edge_index, W_src, b_src, W_dst, b_dst, W_ang, b_ang,
           W_msg, b_msg, W1, b1, W2, b2, gamma, beta):
    ei = edge_index.astype(jnp.int32)
    src, dst = ei[0], ei[1]
    nl2e = jnp.float32(-1.0)
    b_sum = ((b_src + b_dst + b_ang) * nl2e).reshape(1, D)

    e_pad = jnp.concatenate([e, jnp.zeros((NPAD - N, D), _f32)])
    sg_t, sm_t, sd_t = _node_proj(
        e_pad, W_src * nl2e, W_msg, W_dst * nl2e, b_msg.reshape(1, D), b_sum)
    g_t = _ang_proj(a, W_ang * nl2e).reshape(2 * E, H)

    src_p = jnp.concatenate([src, jnp.zeros((EPAD - E,), jnp.int32)])
    dst_p = jnp.concatenate([dst, jnp.full((EPAD - E,), DUMP, jnp.int32)])
    srco = jnp.stack([src_p, src_p + NPAD]).reshape(2, EPAD // CHUNK, CHUNK)
    dsto = jnp.stack([dst_p, dst_p + NPAD]).reshape(2, EPAD // CHUNK, CHUNK)
    dstp = dst_p.reshape(EPAD // CHUNK, CHUNK)

    zeros = jnp.zeros((NPAD, H), _f32)
    agg = _edge_phase(sg_t.reshape(2 * NPAD, H), sm_t.reshape(2 * NPAD, H),
                      sd_t.reshape(2 * NPAD, H), g_t,
                      srco, dsto, dstp, zeros)

    return _node_mlp(e, agg[0, :N], agg[1, :N], W1[:D], W1[D:D + H],
                     W1[D + H:], b1.reshape(1, D), W2, b2.reshape(1, D),
                     gamma.reshape(1, D), beta.reshape(1, D))


# speedup vs baseline: 1.1617x; 1.0620x over previous
import functools

import jax
import jax.numpy as jnp
from jax import lax
from jax.experimental import pallas as pl
from jax.experimental.pallas import tpu as pltpu
from jax.experimental.pallas import tpu_sc as plsc

N = 10000
E = 160000
D = 256
H = D // 2
NC = 2
NS = 16
LANES = 16
NPAD = 10112
EPAD = 163840
EPT = EPAD // NS
CHUNK = 32
NCHUNKS = EPT // CHUNK
MACRO = 8
NM = NCHUNKS // MACRO
BODIES = NCHUNKS // 2
ROWS_PER_SUB = NPAD // NS
GBYTES = 4 * CHUNK * H * 4
DUMP = NPAD - 1

_f32 = jnp.float32


def _proj_body(e_ref, ws_ref, wm_ref, wd_ref, bm_ref, bsum_ref,
               sg_ref, sm_ref, sd_ref):
    e = e_ref[...]
    ps = jnp.dot(e, ws_ref[...], preferred_element_type=_f32)
    pm = jnp.dot(e, wm_ref[...], preferred_element_type=_f32) + bm_ref[...]
    pd = jnp.dot(e, wd_ref[...], preferred_element_type=_f32) + bsum_ref[...]
    sg_ref[0] = ps[:, :H]
    sg_ref[1] = ps[:, H:]
    sm_ref[0] = pm[:, :H]
    sm_ref[1] = pm[:, H:]
    sd_ref[0] = pd[:, :H]
    sd_ref[1] = pd[:, H:]


def _node_proj(e_pad, w_src, w_msg, w_dst, b_msg, b_sum):
    rb = NPAD // 16
    grid = (NPAD // rb,)
    full = pl.BlockSpec((D, D), lambda i: (0, 0))
    vec = pl.BlockSpec((1, D), lambda i: (0, 0))
    return pl.pallas_call(
        _proj_body,
        grid=grid,
        in_specs=[pl.BlockSpec((rb, D), lambda i: (i, 0)), full, full, full,
                  vec, vec],
        out_specs=[pl.BlockSpec((2, rb, H), lambda i: (0, i, 0))] * 3,
        out_shape=[jax.ShapeDtypeStruct((2, NPAD, H), _f32)] * 3,
    )(e_pad, w_src, w_msg, w_dst, b_msg, b_sum)


def _ang_body(a_ref, w_ref, g_ref):
    g = jnp.dot(a_ref[...], w_ref[...], preferred_element_type=_f32)
    g_ref[0] = g[:, :H]
    g_ref[1] = g[:, H:]


def _ang_proj(a, w_ang):
    rb = 2000
    grid = (E // rb,)
    return pl.pallas_call(
        _ang_body,
        grid=grid,
        in_specs=[pl.BlockSpec((rb, D), lambda i: (i, 0)),
                  pl.BlockSpec((D, D), lambda i: (0, 0))],
        out_specs=pl.BlockSpec((2, rb, H), lambda i: (0, i, 0)),
        out_shape=jax.ShapeDtypeStruct((2, E, H), _f32),
    )(a, w_ang)


def _edge_body(sg_t, sm_t, sd_t, g_t, sodo, dstp, zeros_hbm, agg_out,
               sodo0, dp0, sodo1, dp1,
               sgA, smA, sdA, gA, sgB, smB, sdB, gB, mA, mB,
               agg_sh, semA, semB, semS):
    cid = lax.axis_index("c")
    sid = lax.axis_index("s")

    my_rows = pl.ds(sid * ROWS_PER_SUB, ROWS_PER_SUB)
    pltpu.sync_copy(zeros_hbm.at[my_rows], agg_sh.at[my_rows])

    idx_row0 = sid * (EPT // CHUNK)

    def load_macro(m, so_do, dp):
        rb = pl.multiple_of(idx_row0 + m * MACRO, 8)
        pltpu.sync_copy(sodo.at[cid, pl.ds(rb, MACRO)], so_do)
        pltpu.sync_copy(dstp.at[pl.ds(rb, MACRO)], dp)

    def issue(c, sg_b, sm_b, sd_b, g_b, sem, so_do):
        r = lax.rem(c, MACRO)
        so = so_do.at[r, pl.ds(0, CHUNK)]
        do = so_do.at[r, pl.ds(CHUNK, CHUNK)]
        pltpu.async_copy(sg_t.at[so], sg_b, sem)
        pltpu.async_copy(sm_t.at[so], sm_b, sem)
        pltpu.async_copy(sd_t.at[do], sd_b, sem)
        gbase = pl.multiple_of(
            cid * E + jnp.minimum(sid * EPT + c * CHUNK, E - CHUNK), 8)
        pltpu.async_copy(g_t.at[pl.ds(gbase, CHUNK)], g_b, sem)

    def issue_p(c, sg_b, sm_b, sd_b, g_b, sem):
        par = lax.rem(lax.div(c, MACRO), 2)

        @pl.when(par == 0)
        def _():
            issue(c, sg_b, sm_b, sd_b, g_b, sem, sodo0)

        @pl.when(par == 1)
        def _():
            issue(c, sg_b, sm_b, sd_b, g_b, sem, sodo1)

    def drain(sg_b, sm_b, sd_b, g_b, sem):
        dummy = sg_t.at[pl.ds(0, CHUNK)]
        pltpu.make_async_copy(dummy, sg_b, sem).wait()
        pltpu.make_async_copy(dummy, sm_b, sem).wait()
        pltpu.make_async_copy(dummy, sd_b, sem).wait()
        pltpu.make_async_copy(dummy, g_b, sem).wait()

    def drain_scatter(m_b):
        pltpu.make_async_copy(sg_t.at[pl.ds(0, CHUNK)], m_b, semS).wait()

    def compute(sg_b, sm_b, sd_b, g_b, m_b):
        @plsc.parallel_loop(0, CHUNK, unroll=2)
        def _(i):
            for j in range(H // LANES):
                fs = pl.ds(j * LANES, LANES)
                x = sg_b[i, fs] + sd_b[i, fs] + g_b[i, fs]
                m_b[i, fs] = sm_b[i, fs] / (1.0 + jnp.exp(x))

    def scatter(c, m_b):
        r = lax.rem(c, MACRO)
        par = lax.rem(lax.div(c, MACRO), 2)

        @pl.when(par == 0)
        def _():
            pltpu.async_copy(m_b, agg_sh.at[dp0.at[r]], semS, add=True)

        @pl.when(par == 1)
        def _():
            pltpu.async_copy(m_b, agg_sh.at[dp1.at[r]], semS, add=True)

    load_macro(0, sodo0, dp0)
    issue(0, sgA, smA, sdA, gA, semA, sodo0)

    def body(k, carry):
        c0 = 2 * k
        c1 = c0 + 1
        cn = c0 + 2

        issue_p(c1, sgB, smB, sdB, gB, semB)

        @pl.when(lax.rem(k, MACRO // 2) == 0)
        def _():
            mn = jnp.minimum(lax.div(k, MACRO // 2) + 1, NM - 1)

            @pl.when(lax.rem(mn, 2) == 0)
            def _():
                load_macro(mn, sodo0, dp0)

            @pl.when(lax.rem(mn, 2) == 1)
            def _():
                load_macro(mn, sodo1, dp1)

        drain(sgA, smA, sdA, gA, semA)

        @pl.when(k > 0)
        def _():
            drain_scatter(mA)

        compute(sgA, smA, sdA, gA, mA)
        scatter(c0, mA)

        @pl.when(cn < NCHUNKS)
        def _():
            issue_p(cn, sgA, smA, sdA, gA, semA)

        drain(sgB, smB, sdB, gB, semB)

        @pl.when(k > 0)
        def _():
            drain_scatter(mB)

        compute(sgB, smB, sdB, gB, mB)
        scatter(c1, mB)
        return carry

    lax.fori_loop(0, BODIES, body, 0)
    drain_scatter(mA)
    drain_scatter(mB)
    plsc.subcore_barrier()

    pltpu.sync_copy(agg_sh.at[my_rows], agg_out.at[cid, my_rows])


_edge_phase = functools.partial(
    pl.kernel,
    _edge_body,
    out_type=jax.ShapeDtypeStruct((2, NPAD, H), _f32),
    mesh=plsc.VectorSubcoreMesh(core_axis_name="c", subcore_axis_name="s"),
    scratch_types=[
        pltpu.VMEM((MACRO, 2 * CHUNK), jnp.int32),
        pltpu.VMEM((MACRO, CHUNK), jnp.int32),
        pltpu.VMEM((MACRO, 2 * CHUNK), jnp.int32),
        pltpu.VMEM((MACRO, CHUNK), jnp.int32),
        pltpu.VMEM((CHUNK, H), _f32),
        pltpu.VMEM((CHUNK, H), _f32),
        pltpu.VMEM((CHUNK, H), _f32),
        pltpu.VMEM((CHUNK, H), _f32),
        pltpu.VMEM((CHUNK, H), _f32),
        pltpu.VMEM((CHUNK, H), _f32),
        pltpu.VMEM((CHUNK, H), _f32),
        pltpu.VMEM((CHUNK, H), _f32),
        pltpu.VMEM((CHUNK, H), _f32),
        pltpu.VMEM((CHUNK, H), _f32),
        pltpu.VMEM_SHARED((NPAD, H), _f32),
        pltpu.SemaphoreType.DMA,
        pltpu.SemaphoreType.DMA,
        pltpu.SemaphoreType.DMA,
    ],
)()


def _mlp_body(e_ref, a0_ref, a1_ref, w1e_ref, w1a0_ref, w1a1_ref, b1_ref,
              w2_ref, b2_ref, gam_ref, bet_ref, out_ref):
    e = e_ref[...]
    h = (jnp.dot(e, w1e_ref[...], preferred_element_type=_f32)
         + jnp.dot(a0_ref[...], w1a0_ref[...], preferred_element_type=_f32)
         + jnp.dot(a1_ref[...], w1a1_ref[...], preferred_element_type=_f32)
         + b1_ref[...])
    h = h * (1.0 / (1.0 + jnp.exp(-h)))
    h = jnp.dot(h, w2_ref[...], preferred_element_type=_f32) + b2_ref[...]
    x = e + h
    mean = jnp.mean(x, axis=-1, keepdims=True)
    cen = x - mean
    var = jnp.mean(cen * cen, axis=-1, keepdims=True)
    out_ref[...] = cen * lax.rsqrt(var + 1e-5) * gam_ref[...] + bet_ref[...]


def _node_mlp(e, agg0, agg1, w1e, w1a0, w1a1, b1, w2, b2, gamma, beta):
    rb = 1000
    grid = (N // rb,)
    vec = pl.BlockSpec((1, D), lambda i: (0, 0))
    return pl.pallas_call(
        _mlp_body,
        grid=grid,
        in_specs=[pl.BlockSpec((rb, D), lambda i: (i, 0)),
                  pl.BlockSpec((rb, H), lambda i: (i, 0)),
                  pl.BlockSpec((rb, H), lambda i: (i, 0)),
                  pl.BlockSpec((D, D), lambda i: (0, 0)),
                  pl.BlockSpec((H, D), lambda i: (0, 0)),
                  pl.BlockSpec((H, D), lambda i: (0, 0)),
                  vec,
                  pl.BlockSpec((D, D), lambda i: (0, 0)),
                  vec, vec, vec],
        out_specs=pl.BlockSpec((rb, D), lambda i: (i, 0)),
        out_shape=jax.ShapeDtypeStruct((N, D), _f32),
    )(e, agg0, agg1, w1e, w1a0, w1a1, b1, w2, b2, gamma, beta)


def kernel(e, a, edge_index, W_src, b_src, W_dst, b_dst, W_ang, b_ang,
           W_msg, b_msg, W1, b1, W2, b2, gamma, beta):
    ei = edge_index.astype(jnp.int32)
    src, dst = ei[0], ei[1]
    nl2e = jnp.float32(-1.0)
    b_sum = ((b_src + b_dst + b_ang) * nl2e).reshape(1, D)

    e_pad = jnp.concatenate([e, jnp.zeros((NPAD - N, D), _f32)])
    sg_t, sm_t, sd_t = _node_proj(
        e_pad, W_src * nl2e, W_msg, W_dst * nl2e, b_msg.reshape(1, D), b_sum)
    g_t = _ang_proj(a, W_ang * nl2e).reshape(2 * E, H)

    src_p = jnp.concatenate([src, jnp.zeros((EPAD - E,), jnp.int32)])
    dst_p = jnp.concatenate([dst, jnp.full((EPAD - E,), DUMP, jnp.int32)])
    srco = jnp.stack([src_p, src_p + NPAD]).reshape(2, EPAD // CHUNK, CHUNK)
    dsto = jnp.stack([dst_p, dst_p + NPAD]).reshape(2, EPAD // CHUNK, CHUNK)
    sodo = jnp.concatenate([srco, dsto], axis=2)
    dstp = dst_p.reshape(EPAD // CHUNK, CHUNK)

    zeros = jnp.zeros((NPAD, H), _f32)
    agg = _edge_phase(sg_t.reshape(2 * NPAD, H), sm_t.reshape(2 * NPAD, H),
                      sd_t.reshape(2 * NPAD, H), g_t,
                      sodo, dstp, zeros)

    return _node_mlp(e, agg[0, :N], agg[1, :N], W1[:D], W1[D:D + H],
                     W1[D + H:], b1.reshape(1, D), W2, b2.reshape(1, D),
                     gamma.reshape(1, D), beta.reshape(1, D))
